# Initial kernel scaffold; baseline (speedup 1.0000x reference)
import jax
import jax.numpy as jnp
from jax.experimental import pallas as pl


def kernel(x, edge_index, Wc, bc, W1, b1, W2, b2):
    raise NotImplementedError("write your pallas kernel here")


# speedup vs baseline: 19.0084x; 19.0084x over previous
import functools

import jax
import jax.numpy as jnp
from jax import lax
from jax.experimental import pallas as pl
from jax.experimental.pallas import tpu as pltpu
from jax.experimental.pallas import tpu_sc as plsc

N_NODES = 10000
E = 320000
D_IN = 128
D_LAT = 64
N_NEU = 50
FC_IN = N_NEU * N_NEU
FC_HID = 2 * FC_IN
G = N_NODES // N_NEU

NC, NS = 2, 16
NW = NC * NS
CH = 128
NCH = 79
EPW = NCH * CH
EPAD = NW * EPW
NPAD = 10112
RPT = NPAD // NS
NB = 4
NFULL = (NCH // NB) * NB


DW = 16


def _sc_deg_body(dst_hbm, ones_hbm, zeros_hbm, deg_hbm,
                 idx_v, ones_v, zbuf, deg_sh, sem):
    c = lax.axis_index("c")
    s = lax.axis_index("s")
    wid = c * NS + s
    pltpu.sync_copy(ones_hbm, ones_v)
    pltpu.sync_copy(zeros_hbm, zbuf)
    pltpu.sync_copy(dst_hbm.at[pl.ds(wid * NCH, NCH)], idx_v)

    @pl.loop(0, RPT, step=8)
    def _(r):
        pltpu.async_copy(zbuf, deg_sh.at[pl.ds(s * RPT + r, 8)], sem)

    @pl.loop(0, RPT, step=8)
    def _(r):
        pltpu.make_async_copy(zbuf, deg_sh.at[pl.ds(s * RPT + r, 8)],
                              sem).wait()

    plsc.subcore_barrier()

    @pl.loop(0, NCH)
    def _(j):
        pltpu.async_copy(ones_v, deg_sh.at[idx_v.at[j]], sem, add=True)

    @pl.loop(0, NCH)
    def _(j):
        pltpu.make_async_copy(ones_v, deg_sh.at[idx_v.at[j]], sem).wait()

    plsc.subcore_barrier()

    @pl.loop(0, RPT - CH, step=CH)
    def _(r):
        pltpu.sync_copy(deg_sh.at[pl.ds(s * RPT + r, CH)], ones_v)
        pltpu.sync_copy(ones_v, deg_hbm.at[pl.ds(c * NPAD + s * RPT + r, CH)])

    tail = RPT - (RPT // CH) * CH
    base = (RPT // CH) * CH
    tv = ones_v.at[pl.ds(0, tail), pl.ds(0, DW)]
    pltpu.sync_copy(deg_sh.at[pl.ds(s * RPT + base, tail)], tv)
    pltpu.sync_copy(tv, deg_hbm.at[pl.ds(c * NPAD + s * RPT + base, tail)])


def _sc_scatter_body(src_hbm, dst_hbm, hs_hbm, zeros_hbm, acc_hbm,
                     sidx_v, didx_v, r0, r1, r2, r3, zbuf, acc_sh,
                     g0, g1, g2, g3, s0, s1, s2, s3):
    c = lax.axis_index("c")
    s = lax.axis_index("s")
    wid = c * NS + s
    rows = (r0, r1, r2, r3)
    gsem = (g0, g1, g2, g3)
    ssem = (s0, s1, s2, s3)
    pltpu.sync_copy(zeros_hbm, zbuf)
    pltpu.sync_copy(src_hbm.at[pl.ds(wid * NCH, NCH)], sidx_v)
    pltpu.sync_copy(dst_hbm.at[pl.ds(wid * NCH, NCH)], didx_v)

    @pl.loop(0, RPT, step=8)
    def _(r):
        pltpu.async_copy(zbuf, acc_sh.at[pl.ds(s * RPT + r, 8)], g0)

    @pl.loop(0, RPT, step=8)
    def _(r):
        pltpu.make_async_copy(zbuf, acc_sh.at[pl.ds(s * RPT + r, 8)],
                              g0).wait()

    plsc.subcore_barrier()

    @pl.loop(0, NFULL, step=NB)
    def _(j0):
        for b in range(NB):
            jb = j0 + b

            @pl.when(j0 > 0)
            def _():
                pltpu.make_async_copy(rows[b], acc_sh.at[didx_v.at[jb]],
                                      ssem[b]).wait()

            pltpu.async_copy(hs_hbm.at[sidx_v.at[jb]], rows[b], gsem[b])
        for b in range(NB):
            jb = j0 + b
            pltpu.make_async_copy(hs_hbm.at[sidx_v.at[jb]], rows[b],
                                  gsem[b]).wait()
            pltpu.async_copy(rows[b], acc_sh.at[didx_v.at[jb]], ssem[b],
                             add=True)

    for b in range(NB):
        pltpu.make_async_copy(rows[b], acc_sh.at[didx_v.at[NFULL - NB + b]],
                              ssem[b]).wait()
    for j in range(NFULL, NCH):
        pltpu.sync_copy(hs_hbm.at[sidx_v.at[j]], r0)
        pltpu.sync_copy(r0, acc_sh.at[didx_v.at[j]], add=True)

    plsc.subcore_barrier()

    @pl.loop(0, RPT - CH, step=CH)
    def _(r):
        pltpu.sync_copy(acc_sh.at[pl.ds(s * RPT + r, CH)], r0)
        pltpu.sync_copy(r0, acc_hbm.at[pl.ds(c * NPAD + s * RPT + r, CH)])

    tail = RPT - (RPT // CH) * CH
    base = (RPT // CH) * CH
    tview = r1.at[pl.ds(0, tail), pl.ds(0, D_LAT)]
    pltpu.sync_copy(acc_sh.at[pl.ds(s * RPT + base, tail)], tview)
    pltpu.sync_copy(tview, acc_hbm.at[pl.ds(c * NPAD + s * RPT + base, tail)])


@functools.cache
def _sc_kernels():
    vmesh = plsc.VectorSubcoreMesh(core_axis_name="c", subcore_axis_name="s",
                                   num_cores=NC, num_subcores=NS)
    sc_deg = pl.kernel(
        _sc_deg_body,
        out_type=jax.ShapeDtypeStruct((NC * NPAD, DW), jnp.float32),
        mesh=vmesh,
        compiler_params=pltpu.CompilerParams(use_tc_tiling_on_sc=False),
        scratch_types=[
            pltpu.VMEM((NCH, CH), jnp.int32),
            pltpu.VMEM((CH, DW), jnp.float32),
            pltpu.VMEM((8, DW), jnp.float32),
            pltpu.VMEM_SHARED((NPAD, DW), jnp.float32),
            pltpu.SemaphoreType.DMA,
        ],
    )
    sc_scatter = pl.kernel(
        _sc_scatter_body,
        out_type=jax.ShapeDtypeStruct((NC * NPAD, D_LAT), jnp.float32),
        mesh=vmesh,
        compiler_params=pltpu.CompilerParams(use_tc_tiling_on_sc=False),
        scratch_types=(
            [pltpu.VMEM((NCH, CH), jnp.int32)] * 2
            + [pltpu.VMEM((CH, D_LAT), jnp.float32)] * 4
            + [pltpu.VMEM((8, D_LAT), jnp.float32)]
            + [pltpu.VMEM_SHARED((NPAD, D_LAT), jnp.float32)]
            + [pltpu.SemaphoreType.DMA] * 8
        ),
    )
    return sc_deg, sc_scatter


def _k1_body(x_ref, w_ref, o_ref):
    o_ref[...] = jnp.dot(x_ref[...], w_ref[...],
                         preferred_element_type=jnp.float32)


def _k1(x, Wc):
    return pl.pallas_call(
        _k1_body,
        out_shape=jax.ShapeDtypeStruct((N_NODES, D_LAT), jnp.float32),
        grid=(5,),
        in_specs=[
            pl.BlockSpec((N_NODES // 5, D_IN), lambda i: (i, 0)),
            pl.BlockSpec((D_IN, D_LAT), lambda i: (0, 0)),
        ],
        out_specs=pl.BlockSpec((N_NODES // 5, D_LAT), lambda i: (i, 0)),
    )(x, Wc)


def _k1b_body(deg_ref, h_ref, hs_ref, dinv_ref):
    deg = deg_ref[0, :, 0:1] + deg_ref[1, :, 0:1]
    dinv = jnp.where(deg > 0, lax.rsqrt(jnp.maximum(deg, 1e-12)), 0.0)
    dinv_ref[...] = dinv
    hs_ref[0:N_NODES, :] = dinv[0:N_NODES] * h_ref[...]
    hs_ref[N_NODES:NPAD, :] = jnp.zeros((NPAD - N_NODES, D_LAT), jnp.float32)


def _k1b(deg2, h):
    return pl.pallas_call(
        _k1b_body,
        out_shape=(
            jax.ShapeDtypeStruct((NPAD, D_LAT), jnp.float32),
            jax.ShapeDtypeStruct((NPAD, 1), jnp.float32),
        ),
    )(deg2, h)


GB = 8
RB = GB * N_NEU


def _k2a_body(acc_ref, dinv_ref, bc_ref, o_ref):
    accsum = acc_ref[0] + acc_ref[1]
    z = jnp.maximum(dinv_ref[...] * accsum + bc_ref[...], 0.0)
    big = lax.dot_general(z, z, (((1,), (1,)), ((), ())),
                          preferred_element_type=jnp.float32)
    for g in range(GB):
        o_ref[g] = big[g * N_NEU:(g + 1) * N_NEU, g * N_NEU:(g + 1) * N_NEU]


def _k2a(acc2, dinv, bc_row):
    return pl.pallas_call(
        _k2a_body,
        out_shape=jax.ShapeDtypeStruct((G, N_NEU, N_NEU), jnp.float32),
        grid=(G // GB,),
        in_specs=[
            pl.BlockSpec((NC, RB, D_LAT), lambda i: (0, i, 0)),
            pl.BlockSpec((RB, 1), lambda i: (i, 0)),
            pl.BlockSpec((1, D_LAT), lambda i: (0, 0)),
        ],
        out_specs=pl.BlockSpec((GB, N_NEU, N_NEU), lambda i: (i, 0, 0)),
    )(acc2, dinv, bc_row)


def _make_fct_body(act):
    def body(xt_ref, w_ref, b_ref, o_ref):
        @pl.when(pl.program_id(0) == 0)
        def _():
            o_ref[...] = jnp.broadcast_to(b_ref[...], o_ref.shape)

        o_ref[...] += lax.dot_general(
            w_ref[0], xt_ref[0], (((0,), (0,)), ((), ())),
            preferred_element_type=jnp.float32)

        @pl.when(pl.program_id(0) == pl.num_programs(0) - 1)
        def _():
            o_ref[...] = act(o_ref[...])

    return body


_fc_relu_body = _make_fct_body(lambda v: jnp.maximum(v, 0.0))
_fc_sig_body = _make_fct_body(jax.nn.sigmoid)


def _fc(body, xt, w, b_col, nk):
    k, m = xt.shape
    n = w.shape[1]
    kblk = k // nk
    xt3 = xt.reshape(nk, kblk, m)
    w3 = w.reshape(nk, kblk, n)
    return pl.pallas_call(
        body,
        out_shape=jax.ShapeDtypeStruct((n, m), jnp.float32),
        grid=(nk,),
        in_specs=[
            pl.BlockSpec((1, kblk, m), lambda i: (i, 0, 0)),
            pl.BlockSpec((1, kblk, n), lambda i: (i, 0, 0)),
            pl.BlockSpec((n, 1), lambda i: (0, 0)),
        ],
        out_specs=pl.BlockSpec((n, m), lambda i: (0, 0)),
    )(xt3, w3, b_col)


def kernel(x, edge_index, Wc, bc, W1, b1, W2, b2):
    ei = edge_index.astype(jnp.int32)
    pad = N_NODES + (jnp.arange(EPAD - E, dtype=jnp.int32) % 16)
    src2 = jnp.concatenate([ei[0], pad]).reshape(NW * NCH, CH)
    dst3 = jnp.concatenate([ei[1], pad]).reshape(NW, NCH, CH)
    dst2 = dst3.reshape(NW * NCH, CH)
    ones_c = jnp.ones((CH, DW), jnp.float32)
    zer1 = jnp.zeros((8, DW), jnp.float32)
    zer64 = jnp.zeros((8, D_LAT), jnp.float32)

    sc_deg, sc_scatter = _sc_kernels()
    deg2 = sc_deg(dst2, ones_c, zer1)
    h = _k1(x, Wc)
    hs, dinv = _k1b(deg2.reshape(NC, NPAD, DW), h)
    acc2 = sc_scatter(src2, dst2, hs, zer64)
    gf = _k2a(acc2.reshape(NC, NPAD, D_LAT), dinv,
              bc.reshape(1, D_LAT))
    gft = gf.reshape(G, FC_IN).T
    at = _fc(_fc_relu_body, gft, W1, b1.reshape(FC_HID, 1), 4)
    yt = _fc(_fc_sig_body, at, W2, b2.reshape(FC_IN, 1), 4)
    return yt.T.reshape(-1)


# speedup vs baseline: 22.0657x; 1.1608x over previous
import functools

import jax
import jax.numpy as jnp
from jax import lax
from jax.experimental import pallas as pl
from jax.experimental.pallas import tpu as pltpu
from jax.experimental.pallas import tpu_sc as plsc

N_NODES = 10000
E = 320000
D_IN = 128
D_LAT = 64
N_NEU = 50
FC_IN = N_NEU * N_NEU
FC_HID = 2 * FC_IN
G = N_NODES // N_NEU

NC, NS = 2, 16
NW = NC * NS
CH = 128
NCH = 79
EPW = NCH * CH
EPAD = NW * EPW
NPAD = 10112
RPT = NPAD // NS
NB = 4
NFULL = (NCH // NB) * NB


DW = 16


def _sc_deg_body(dst_hbm, ones_hbm, zeros_hbm, deg_hbm,
                 idx_v, ones_v, zbuf, deg_sh, sem):
    c = lax.axis_index("c")
    s = lax.axis_index("s")
    wid = c * NS + s
    pltpu.sync_copy(ones_hbm, ones_v)
    pltpu.sync_copy(zeros_hbm, zbuf)
    pltpu.sync_copy(dst_hbm.at[pl.ds(wid * NCH, NCH)], idx_v)

    @pl.loop(0, RPT, step=8)
    def _(r):
        pltpu.async_copy(zbuf, deg_sh.at[pl.ds(s * RPT + r, 8)], sem)

    @pl.loop(0, RPT, step=8)
    def _(r):
        pltpu.make_async_copy(zbuf, deg_sh.at[pl.ds(s * RPT + r, 8)],
                              sem).wait()

    plsc.subcore_barrier()

    @pl.loop(0, NCH)
    def _(j):
        pltpu.async_copy(ones_v, deg_sh.at[idx_v.at[j]], sem, add=True)

    @pl.loop(0, NCH)
    def _(j):
        pltpu.make_async_copy(ones_v, deg_sh.at[idx_v.at[j]], sem).wait()

    plsc.subcore_barrier()

    @pl.loop(0, RPT - CH, step=CH)
    def _(r):
        pltpu.sync_copy(deg_sh.at[pl.ds(s * RPT + r, CH)], ones_v)
        pltpu.sync_copy(ones_v, deg_hbm.at[pl.ds(c * NPAD + s * RPT + r, CH)])

    tail = RPT - (RPT // CH) * CH
    base = (RPT // CH) * CH
    tv = ones_v.at[pl.ds(0, tail), pl.ds(0, DW)]
    pltpu.sync_copy(deg_sh.at[pl.ds(s * RPT + base, tail)], tv)
    pltpu.sync_copy(tv, deg_hbm.at[pl.ds(c * NPAD + s * RPT + base, tail)])


def _sc_scatter_body(src_hbm, dst_hbm, hs_hbm, zeros_hbm, acc_hbm,
                     sidx_v, didx_v, r0, r1, r2, r3, zbuf, acc_sh,
                     g0, g1, g2, g3, s0, s1, s2, s3):
    c = lax.axis_index("c")
    s = lax.axis_index("s")
    wid = c * NS + s
    rows = (r0, r1, r2, r3)
    gsem = (g0, g1, g2, g3)
    ssem = (s0, s1, s2, s3)
    pltpu.sync_copy(zeros_hbm, zbuf)
    pltpu.sync_copy(src_hbm.at[pl.ds(wid * NCH, NCH)], sidx_v)
    pltpu.sync_copy(dst_hbm.at[pl.ds(wid * NCH, NCH)], didx_v)

    @pl.loop(0, RPT, step=8)
    def _(r):
        pltpu.async_copy(zbuf, acc_sh.at[pl.ds(s * RPT + r, 8)], g0)

    @pl.loop(0, RPT, step=8)
    def _(r):
        pltpu.make_async_copy(zbuf, acc_sh.at[pl.ds(s * RPT + r, 8)],
                              g0).wait()

    plsc.subcore_barrier()

    @pl.loop(0, NFULL, step=NB)
    def _(j0):
        for b in range(NB):
            jb = j0 + b

            @pl.when(j0 > 0)
            def _():
                pltpu.make_async_copy(rows[b], acc_sh.at[didx_v.at[jb]],
                                      ssem[b]).wait()

            pltpu.async_copy(hs_hbm.at[sidx_v.at[jb]], rows[b], gsem[b])
        for b in range(NB):
            jb = j0 + b
            pltpu.make_async_copy(hs_hbm.at[sidx_v.at[jb]], rows[b],
                                  gsem[b]).wait()
            pltpu.async_copy(rows[b], acc_sh.at[didx_v.at[jb]], ssem[b],
                             add=True)

    for b in range(NB):
        pltpu.make_async_copy(rows[b], acc_sh.at[didx_v.at[NFULL - NB + b]],
                              ssem[b]).wait()
    for j in range(NFULL, NCH):
        pltpu.sync_copy(hs_hbm.at[sidx_v.at[j]], r0)
        pltpu.sync_copy(r0, acc_sh.at[didx_v.at[j]], add=True)

    plsc.subcore_barrier()

    @pl.loop(0, RPT - CH, step=CH)
    def _(r):
        pltpu.sync_copy(acc_sh.at[pl.ds(s * RPT + r, CH)], r0)
        pltpu.sync_copy(r0, acc_hbm.at[pl.ds(c * NPAD + s * RPT + r, CH)])

    tail = RPT - (RPT // CH) * CH
    base = (RPT // CH) * CH
    tview = r1.at[pl.ds(0, tail), pl.ds(0, D_LAT)]
    pltpu.sync_copy(acc_sh.at[pl.ds(s * RPT + base, tail)], tview)
    pltpu.sync_copy(tview, acc_hbm.at[pl.ds(c * NPAD + s * RPT + base, tail)])


@functools.cache
def _sc_kernels():
    vmesh = plsc.VectorSubcoreMesh(core_axis_name="c", subcore_axis_name="s",
                                   num_cores=NC, num_subcores=NS)
    sc_deg = pl.kernel(
        _sc_deg_body,
        out_type=jax.ShapeDtypeStruct((NC * NPAD, DW), jnp.float32),
        mesh=vmesh,
        compiler_params=pltpu.CompilerParams(use_tc_tiling_on_sc=False),
        scratch_types=[
            pltpu.VMEM((NCH, CH), jnp.int32),
            pltpu.VMEM((CH, DW), jnp.float32),
            pltpu.VMEM((8, DW), jnp.float32),
            pltpu.VMEM_SHARED((NPAD, DW), jnp.float32),
            pltpu.SemaphoreType.DMA,
        ],
    )
    sc_scatter = pl.kernel(
        _sc_scatter_body,
        out_type=jax.ShapeDtypeStruct((NC * NPAD, D_LAT), jnp.float32),
        mesh=vmesh,
        compiler_params=pltpu.CompilerParams(use_tc_tiling_on_sc=False),
        scratch_types=(
            [pltpu.VMEM((NCH, CH), jnp.int32)] * 2
            + [pltpu.VMEM((CH, D_LAT), jnp.float32)] * 4
            + [pltpu.VMEM((8, D_LAT), jnp.float32)]
            + [pltpu.VMEM_SHARED((NPAD, D_LAT), jnp.float32)]
            + [pltpu.SemaphoreType.DMA] * 8
        ),
    )
    return sc_deg, sc_scatter


def _k1_body(x_ref, w_ref, o_ref):
    o_ref[...] = jnp.dot(x_ref[...], w_ref[...],
                         preferred_element_type=jnp.float32)


def _k1(x, Wc):
    return pl.pallas_call(
        _k1_body,
        out_shape=jax.ShapeDtypeStruct((N_NODES, D_LAT), jnp.float32),
        grid=(5,),
        in_specs=[
            pl.BlockSpec((N_NODES // 5, D_IN), lambda i: (i, 0)),
            pl.BlockSpec((D_IN, D_LAT), lambda i: (0, 0)),
        ],
        out_specs=pl.BlockSpec((N_NODES // 5, D_LAT), lambda i: (i, 0)),
    )(x, Wc)


def _k1b_body(deg_ref, h_ref, hs_ref, dinv_ref):
    deg = deg_ref[0, :, 0:1] + deg_ref[1, :, 0:1]
    dinv = jnp.where(deg > 0, lax.rsqrt(jnp.maximum(deg, 1e-12)), 0.0)
    dinv_ref[...] = dinv
    hs_ref[0:N_NODES, :] = dinv[0:N_NODES] * h_ref[...]
    hs_ref[N_NODES:NPAD, :] = jnp.zeros((NPAD - N_NODES, D_LAT), jnp.float32)


def _k1b(deg2, h):
    return pl.pallas_call(
        _k1b_body,
        out_shape=(
            jax.ShapeDtypeStruct((NPAD, D_LAT), jnp.float32),
            jax.ShapeDtypeStruct((NPAD, 1), jnp.float32),
        ),
    )(deg2, h)


GB = 8
RB = GB * N_NEU


def _k2a_body(acc_ref, dinv_ref, bc_ref, o_ref):
    accsum = acc_ref[0] + acc_ref[1]
    z = jnp.maximum(dinv_ref[...] * accsum + bc_ref[...], 0.0)
    big = lax.dot_general(z, z, (((1,), (1,)), ((), ())),
                          preferred_element_type=jnp.float32)
    for g in range(GB):
        o_ref[g] = big[g * N_NEU:(g + 1) * N_NEU, g * N_NEU:(g + 1) * N_NEU]


def _k2a(acc2, dinv, bc_row):
    return pl.pallas_call(
        _k2a_body,
        out_shape=jax.ShapeDtypeStruct((G, N_NEU, N_NEU), jnp.float32),
        grid=(G // GB,),
        in_specs=[
            pl.BlockSpec((NC, RB, D_LAT), lambda i: (0, i, 0)),
            pl.BlockSpec((RB, 1), lambda i: (i, 0)),
            pl.BlockSpec((1, D_LAT), lambda i: (0, 0)),
        ],
        out_specs=pl.BlockSpec((GB, N_NEU, N_NEU), lambda i: (i, 0, 0)),
    )(acc2, dinv, bc_row)


def _make_fct_body(act):
    def body(xt_ref, w_ref, b_ref, o_ref):
        @pl.when(pl.program_id(0) == 0)
        def _():
            o_ref[...] = jnp.broadcast_to(b_ref[...], o_ref.shape)

        o_ref[...] += lax.dot_general(
            w_ref[0], xt_ref[0], (((0,), (0,)), ((), ())),
            preferred_element_type=jnp.float32)

        @pl.when(pl.program_id(0) == pl.num_programs(0) - 1)
        def _():
            o_ref[...] = act(o_ref[...])

    return body


_relu = lambda v: jnp.maximum(v, 0.0)


def _fc(act, xt3, w3, b_col):
    nk, kblk, m = xt3.shape
    n = w3.shape[2]
    return pl.pallas_call(
        _make_fct_body(act),
        out_shape=jax.ShapeDtypeStruct((n, m), jnp.float32),
        grid=(nk,),
        in_specs=[
            pl.BlockSpec((1, kblk, m), lambda i: (i, 0, 0)),
            pl.BlockSpec((1, kblk, n), lambda i: (i, 0, 0)),
            pl.BlockSpec((n, 1), lambda i: (0, 0)),
        ],
        out_specs=pl.BlockSpec((n, m), lambda i: (0, 0)),
    )(xt3, w3, b_col)


def kernel(x, edge_index, Wc, bc, W1, b1, W2, b2):
    ei = edge_index.astype(jnp.int32)
    pad = N_NODES + (jnp.arange(EPAD - E, dtype=jnp.int32) % 16)
    src2 = jnp.concatenate([ei[0], pad]).reshape(NW * NCH, CH)
    dst3 = jnp.concatenate([ei[1], pad]).reshape(NW, NCH, CH)
    dst2 = dst3.reshape(NW * NCH, CH)
    ones_c = jnp.ones((CH, DW), jnp.float32)
    zer1 = jnp.zeros((8, DW), jnp.float32)
    zer64 = jnp.zeros((8, D_LAT), jnp.float32)

    sc_deg, sc_scatter = _sc_kernels()
    deg2 = sc_deg(dst2, ones_c, zer1)
    h = _k1(x, Wc)
    hs, dinv = _k1b(deg2.reshape(NC, NPAD, DW), h)
    acc2 = sc_scatter(src2, dst2, hs, zer64)
    gf = _k2a(acc2.reshape(NC, NPAD, D_LAT), dinv,
              bc.reshape(1, D_LAT))
    gft = gf.reshape(G, FC_IN).T
    xt3 = gft.astype(jnp.bfloat16).reshape(4, FC_IN // 4, G)
    w13 = W1.astype(jnp.bfloat16).reshape(4, FC_IN // 4, FC_HID)
    at = _fc(_relu, xt3, w13, b1.reshape(FC_HID, 1))
    at3 = at.reshape(5, FC_HID // 5, G)
    w23 = W2.reshape(5, FC_HID // 5, FC_IN)
    yt = _fc(jax.nn.sigmoid, at3, w23, b2.reshape(FC_IN, 1))
    return yt.T.reshape(-1)


# speedup vs baseline: 25.8406x; 1.1711x over previous
import functools

import jax
import jax.numpy as jnp
from jax import lax
from jax.experimental import pallas as pl
from jax.experimental.pallas import tpu as pltpu
from jax.experimental.pallas import tpu_sc as plsc

N_NODES = 10000
E = 320000
D_IN = 128
D_LAT = 64
N_NEU = 50
FC_IN = N_NEU * N_NEU
FC_HID = 2 * FC_IN
G = N_NODES // N_NEU

NC, NS = 2, 16
NW = NC * NS
CH = 128
NCH = 79
EPW = NCH * CH
EPAD = NW * EPW
NPAD = 10112
RPT = NPAD // NS
NB = 4
NFULL = (NCH // NB) * NB


DW = 16


def _sc_deg_body(dst_hbm, ones_hbm, zeros_hbm, deg_hbm,
                 idx_v, ones_v, zbuf, deg_sh, sem):
    c = lax.axis_index("c")
    s = lax.axis_index("s")
    wid = c * NS + s
    pltpu.sync_copy(ones_hbm, ones_v)
    pltpu.sync_copy(zeros_hbm, zbuf)
    pltpu.sync_copy(dst_hbm.at[pl.ds(wid * NCH, NCH)], idx_v)

    @pl.loop(0, RPT, step=8)
    def _(r):
        pltpu.async_copy(zbuf, deg_sh.at[pl.ds(s * RPT + r, 8)], sem)

    @pl.loop(0, RPT, step=8)
    def _(r):
        pltpu.make_async_copy(zbuf, deg_sh.at[pl.ds(s * RPT + r, 8)],
                              sem).wait()

    plsc.subcore_barrier()

    @pl.loop(0, NCH)
    def _(j):
        pltpu.async_copy(ones_v, deg_sh.at[idx_v.at[j]], sem, add=True)

    @pl.loop(0, NCH)
    def _(j):
        pltpu.make_async_copy(ones_v, deg_sh.at[idx_v.at[j]], sem).wait()

    plsc.subcore_barrier()

    @pl.loop(0, RPT - CH, step=CH)
    def _(r):
        pltpu.sync_copy(deg_sh.at[pl.ds(s * RPT + r, CH)], ones_v)
        pltpu.sync_copy(ones_v, deg_hbm.at[pl.ds(c * NPAD + s * RPT + r, CH)])

    tail = RPT - (RPT // CH) * CH
    base = (RPT // CH) * CH
    tv = ones_v.at[pl.ds(0, tail), pl.ds(0, DW)]
    pltpu.sync_copy(deg_sh.at[pl.ds(s * RPT + base, tail)], tv)
    pltpu.sync_copy(tv, deg_hbm.at[pl.ds(c * NPAD + s * RPT + base, tail)])


def _sc_scatter_body(src_hbm, dst_hbm, hs_hbm, zeros_hbm, acc_hbm,
                     sidx_v, didx_v, r0, r1, r2, r3, zbuf, acc_sh,
                     g0, g1, g2, g3, s0, s1, s2, s3):
    c = lax.axis_index("c")
    s = lax.axis_index("s")
    wid = c * NS + s
    rows = (r0, r1, r2, r3)
    gsem = (g0, g1, g2, g3)
    ssem = (s0, s1, s2, s3)
    pltpu.sync_copy(zeros_hbm, zbuf)
    pltpu.sync_copy(src_hbm.at[pl.ds(wid * NCH, NCH)], sidx_v)
    pltpu.sync_copy(dst_hbm.at[pl.ds(wid * NCH, NCH)], didx_v)

    @pl.loop(0, RPT, step=8)
    def _(r):
        pltpu.async_copy(zbuf, acc_sh.at[pl.ds(s * RPT + r, 8)], g0)

    @pl.loop(0, RPT, step=8)
    def _(r):
        pltpu.make_async_copy(zbuf, acc_sh.at[pl.ds(s * RPT + r, 8)],
                              g0).wait()

    plsc.subcore_barrier()

    @pl.loop(0, NFULL, step=NB)
    def _(j0):
        for b in range(NB):
            jb = j0 + b

            @pl.when(j0 > 0)
            def _():
                pltpu.make_async_copy(rows[b], acc_sh.at[didx_v.at[jb]],
                                      ssem[b]).wait()

            pltpu.async_copy(hs_hbm.at[sidx_v.at[jb]], rows[b], gsem[b])
        for b in range(NB):
            jb = j0 + b
            pltpu.make_async_copy(hs_hbm.at[sidx_v.at[jb]], rows[b],
                                  gsem[b]).wait()
            pltpu.async_copy(rows[b], acc_sh.at[didx_v.at[jb]], ssem[b],
                             add=True)

    for b in range(NB):
        pltpu.make_async_copy(rows[b], acc_sh.at[didx_v.at[NFULL - NB + b]],
                              ssem[b]).wait()
    for j in range(NFULL, NCH):
        pltpu.sync_copy(hs_hbm.at[sidx_v.at[j]], r0)
        pltpu.sync_copy(r0, acc_sh.at[didx_v.at[j]], add=True)

    plsc.subcore_barrier()

    @pl.loop(0, RPT - CH, step=CH)
    def _(r):
        pltpu.sync_copy(acc_sh.at[pl.ds(s * RPT + r, CH)], r0)
        pltpu.sync_copy(r0, acc_hbm.at[pl.ds(c * NPAD + s * RPT + r, CH)])

    tail = RPT - (RPT // CH) * CH
    base = (RPT // CH) * CH
    tview = r1.at[pl.ds(0, tail), pl.ds(0, D_LAT)]
    pltpu.sync_copy(acc_sh.at[pl.ds(s * RPT + base, tail)], tview)
    pltpu.sync_copy(tview, acc_hbm.at[pl.ds(c * NPAD + s * RPT + base, tail)])


@functools.cache
def _sc_kernels():
    vmesh = plsc.VectorSubcoreMesh(core_axis_name="c", subcore_axis_name="s",
                                   num_cores=NC, num_subcores=NS)
    sc_deg = pl.kernel(
        _sc_deg_body,
        out_type=jax.ShapeDtypeStruct((NC * NPAD, DW), jnp.float32),
        mesh=vmesh,
        compiler_params=pltpu.CompilerParams(use_tc_tiling_on_sc=False),
        scratch_types=[
            pltpu.VMEM((NCH, CH), jnp.int32),
            pltpu.VMEM((CH, DW), jnp.float32),
            pltpu.VMEM((8, DW), jnp.float32),
            pltpu.VMEM_SHARED((NPAD, DW), jnp.float32),
            pltpu.SemaphoreType.DMA,
        ],
    )
    sc_scatter = pl.kernel(
        _sc_scatter_body,
        out_type=jax.ShapeDtypeStruct((NC * NPAD, D_LAT), jnp.float32),
        mesh=vmesh,
        compiler_params=pltpu.CompilerParams(use_tc_tiling_on_sc=False),
        scratch_types=(
            [pltpu.VMEM((NCH, CH), jnp.int32)] * 2
            + [pltpu.VMEM((CH, D_LAT), jnp.float32)] * 4
            + [pltpu.VMEM((8, D_LAT), jnp.float32)]
            + [pltpu.VMEM_SHARED((NPAD, D_LAT), jnp.float32)]
            + [pltpu.SemaphoreType.DMA] * 8
        ),
    )
    return sc_deg, sc_scatter


def _k1_body(x_ref, w_ref, o_ref):
    o_ref[...] = jnp.dot(x_ref[...], w_ref[...],
                         preferred_element_type=jnp.float32)


def _k1(x, Wc):
    return pl.pallas_call(
        _k1_body,
        out_shape=jax.ShapeDtypeStruct((N_NODES, D_LAT), jnp.float32),
        grid=(5,),
        in_specs=[
            pl.BlockSpec((N_NODES // 5, D_IN), lambda i: (i, 0)),
            pl.BlockSpec((D_IN, D_LAT), lambda i: (0, 0)),
        ],
        out_specs=pl.BlockSpec((N_NODES // 5, D_LAT), lambda i: (i, 0)),
    )(x, Wc)


def _k1b_body(deg_ref, h_ref, hs_ref, dinv_ref):
    deg = deg_ref[0, :, 0:1] + deg_ref[1, :, 0:1]
    dinv = jnp.where(deg > 0, lax.rsqrt(jnp.maximum(deg, 1e-12)), 0.0)
    dinv_ref[...] = dinv
    hs_ref[0:N_NODES, :] = dinv[0:N_NODES] * h_ref[...]
    hs_ref[N_NODES:NPAD, :] = jnp.zeros((NPAD - N_NODES, D_LAT), jnp.float32)


def _k1b(deg2, h):
    return pl.pallas_call(
        _k1b_body,
        out_shape=(
            jax.ShapeDtypeStruct((NPAD, D_LAT), jnp.float32),
            jax.ShapeDtypeStruct((NPAD, 1), jnp.float32),
        ),
    )(deg2, h)


GB = 8
RB = GB * N_NEU


def _k2a_body(acc_ref, dinv_ref, bc_ref, o_ref):
    accsum = acc_ref[0] + acc_ref[1]
    z = jnp.maximum(dinv_ref[...] * accsum + bc_ref[...], 0.0)
    big = lax.dot_general(z, z, (((1,), (1,)), ((), ())),
                          preferred_element_type=jnp.float32)
    for g in range(GB):
        o_ref[g] = big[g * N_NEU:(g + 1) * N_NEU, g * N_NEU:(g + 1) * N_NEU]


def _k2a(acc2, dinv, bc_row):
    return pl.pallas_call(
        _k2a_body,
        out_shape=jax.ShapeDtypeStruct((G, N_NEU, N_NEU), jnp.float32),
        grid=(G // GB,),
        in_specs=[
            pl.BlockSpec((NC, RB, D_LAT), lambda i: (0, i, 0)),
            pl.BlockSpec((RB, 1), lambda i: (i, 0)),
            pl.BlockSpec((1, D_LAT), lambda i: (0, 0)),
        ],
        out_specs=pl.BlockSpec((GB, N_NEU, N_NEU), lambda i: (i, 0, 0)),
    )(acc2, dinv, bc_row)


def _make_fc_body(act):
    def body(x_ref, w_hbm, b_ref, o_ref, wbuf, sem):
        cp = pltpu.make_async_copy(w_hbm, wbuf, sem)
        cp.start()
        cp.wait()
        a = jnp.dot(x_ref[...], wbuf[...], preferred_element_type=jnp.float32)
        o_ref[...] = act(a + b_ref[...])

    return body


def _fc(act, x, wb, b_row):
    m, k = x.shape
    n = wb.shape[1]
    return pl.pallas_call(
        _make_fc_body(act),
        out_shape=jax.ShapeDtypeStruct((m, n), jnp.float32),
        in_specs=[
            pl.BlockSpec((m, k), lambda: (0, 0)),
            pl.BlockSpec(memory_space=pl.ANY),
            pl.BlockSpec((1, n), lambda: (0, 0)),
        ],
        out_specs=pl.BlockSpec((m, n), lambda: (0, 0)),
        scratch_shapes=[
            pltpu.VMEM((k, n), jnp.bfloat16),
            pltpu.SemaphoreType.DMA,
        ],
    )(x, wb, b_row)


_relu = lambda v: jnp.maximum(v, 0.0)


def kernel(x, edge_index, Wc, bc, W1, b1, W2, b2):
    ei = edge_index.astype(jnp.int32)
    pad = N_NODES + (jnp.arange(EPAD - E, dtype=jnp.int32) % 16)
    src2 = jnp.concatenate([ei[0], pad]).reshape(NW * NCH, CH)
    dst3 = jnp.concatenate([ei[1], pad]).reshape(NW, NCH, CH)
    dst2 = dst3.reshape(NW * NCH, CH)
    ones_c = jnp.ones((CH, DW), jnp.float32)
    zer1 = jnp.zeros((8, DW), jnp.float32)
    zer64 = jnp.zeros((8, D_LAT), jnp.float32)

    sc_deg, sc_scatter = _sc_kernels()
    deg2 = sc_deg(dst2, ones_c, zer1)
    h = _k1(x, Wc)
    hs, dinv = _k1b(deg2.reshape(NC, NPAD, DW), h)
    acc2 = sc_scatter(src2, dst2, hs, zer64)
    gf = _k2a(acc2.reshape(NC, NPAD, D_LAT), dinv,
              bc.reshape(1, D_LAT))
    gfb = gf.reshape(G, FC_IN).astype(jnp.bfloat16)
    a = _fc(_relu, gfb, W1.astype(jnp.bfloat16),
            b1.reshape(1, FC_HID))
    y = _fc(jax.nn.sigmoid, a.astype(jnp.bfloat16),
            W2.astype(jnp.bfloat16), b2.reshape(1, FC_IN))
    return y.reshape(-1)
